# baseline (speedup 1.0000x reference)
import functools

import jax
import jax.numpy as jnp
from jax import lax
from jax.experimental import pallas as pl
from jax.experimental.pallas import tpu as pltpu, tpu_sc as plsc

B = 256
LATENT = 50
BUF = 500
BUFP = 512
K = 3
NA = 6
NAP = 8
HID = 512
BT = 32
G1 = 21
G1C = 24
G3 = 11
G3C = 16


def _enc_body(xq_ref, w1_ref, w2_ref, w3_ref, b1_ref, b2_ref, b3_ref, o_ref):
    xqm = xq_ref[...].reshape(BT * G1 * G1C, 64)
    d = [[None, None], [None, None]]
    for ia in range(2):
        for ib in range(2):
            d[ia][ib] = jax.lax.dot_general(
                xqm, w1_ref[ia * 2 + ib], (((1,), (0,)), ((), ())),
                preferred_element_type=jnp.float32).reshape(BT, G1, G1C, 128)
    y = d[1][1]
    y = y + jnp.pad(d[0][1][:, :G1 - 1], ((0, 0), (1, 0), (0, 0), (0, 0)))
    y = y + jnp.pad(d[1][0][:, :, :G1C - 1], ((0, 0), (0, 0), (1, 0), (0, 0)))
    y = y + jnp.pad(d[0][0][:, :G1 - 1, :G1C - 1],
                    ((0, 0), (1, 0), (1, 0), (0, 0)))
    y = jnp.maximum(y + b1_ref[...].reshape(1, 1, 1, 128), 0.0)
    ri = jax.lax.broadcasted_iota(jnp.int32, (1, G1, G1C, 128), 1)
    ci = jax.lax.broadcasted_iota(jnp.int32, (1, G1, G1C, 128), 2)
    li = jax.lax.broadcasted_iota(jnp.int32, (1, G1, G1C, 128), 3)
    lp = li // 64
    lq = (li // 32) % 2
    bad = ((lp == 0) & (ri == 0)) | ((lp == 1) & (ri == G1 - 1)) \
        | ((lq == 0) & (ci == 0)) | ((lq == 1) & (ci >= 20))
    xs2 = jnp.where(bad, 0.0, y)

    xs2m = xs2.reshape(BT * G1 * G1C, 128)
    for ia in range(2):
        for ib in range(2):
            d[ia][ib] = jax.lax.dot_general(
                xs2m, w2_ref[ia * 2 + ib], (((1,), (0,)), ((), ())),
                preferred_element_type=jnp.float32).reshape(BT, G1, G1C, 64)
    y2 = (d[0][0][:, :20, :20] + d[0][1][:, :20, 1:21]
          + d[1][0][:, 1:21, :20] + d[1][1][:, 1:21, 1:21])
    y2 = jnp.maximum(y2 + b2_ref[...].reshape(1, 1, 1, 64), 0.0)

    yr = y2.reshape(BT, 10, 2, 20, 64)
    er, orr = yr[:, :, 0], yr[:, :, 1]
    zrow = jnp.zeros((BT, 1, 20, 64), jnp.float32)
    r0 = jnp.concatenate([zrow, orr], axis=1)
    r1 = jnp.concatenate([er, zrow], axis=1)
    xs3p = []
    zc1 = jnp.zeros((BT, G3, 1, 64), jnp.float32)
    zc5 = jnp.zeros((BT, G3, 5, 64), jnp.float32)
    for rsel in (r0, r1):
        rc = rsel.reshape(BT, G3, 10, 2, 64)
        ec, oc = rc[:, :, :, 0], rc[:, :, :, 1]
        xs3p.append(jnp.concatenate([zc1, oc, zc5], axis=2))
        xs3p.append(jnp.concatenate([ec, zc1, zc5], axis=2))
    xs3 = jnp.concatenate(xs3p, axis=-1)

    xs3m = xs3.reshape(BT * G3 * G3C, 256)
    for ia in range(2):
        for ib in range(2):
            d[ia][ib] = jax.lax.dot_general(
                xs3m, w3_ref[ia * 2 + ib], (((1,), (0,)), ((), ())),
                preferred_element_type=jnp.float32).reshape(BT, G3, G3C, 128)
    y3 = (d[0][0][:, :10, :10] + d[0][1][:, :10, 1:11]
          + d[1][0][:, 1:11, :10] + d[1][1][:, 1:11, 1:11])
    y3 = jnp.maximum(y3 + b3_ref[...].reshape(1, 1, 1, 128), 0.0)
    o_ref[...] = y3


def _pack_w1(W1):
    ws = [[jnp.zeros((64, 128), jnp.float32) for _ in range(2)]
          for _ in range(2)]
    for P in range(2):
        for Q in range(2):
            for kh in range(3):
                dr = 2 * P + kh - 2
                ia = 1 if dr >= 0 else 0
                r4 = dr % 4
                for kw in range(3):
                    dc = 2 * Q + kw - 2
                    ib = 1 if dc >= 0 else 0
                    c4 = dc % 4
                    blk = W1[:, :, kh, kw].T
                    row = r4 * 16 + c4 * 4
                    col = (P * 2 + Q) * 32
                    ws[ia][ib] = jax.lax.dynamic_update_slice(
                        ws[ia][ib], blk, (row, col))
    return jnp.stack([ws[0][0], ws[0][1], ws[1][0], ws[1][1]])


def _pack_w23(W, cin, cout):
    ws = [[jnp.zeros((4 * cin, cout), jnp.float32) for _ in range(2)]
          for _ in range(2)]
    for kh in range(3):
        a, p = kh // 2, kh % 2 if kh < 2 else 0
        if kh == 2:
            a, p = 1, 0
        for kw in range(3):
            b, q = (1, 0) if kw == 2 else (0, kw)
            blk = W[:, :, kh, kw].T
            ws[a][b] = jax.lax.dynamic_update_slice(
                ws[a][b], blk, ((p * 2 + q) * cin, 0))
    return jnp.stack([ws[0][0], ws[0][1], ws[1][0], ws[1][1]])


def _tail_body(h3_ref, wfc_ref, bfc_ref, g_ref, be_ref, buf_ref, o_ref):
    h = jax.lax.dot_general(h3_ref[...], wfc_ref[...], (((1,), (0,)), ((), ())),
                            preferred_element_type=jnp.float32) + bfc_ref[...]
    mu = jnp.mean(h, axis=1, keepdims=True)
    var = jnp.mean((h - mu) * (h - mu), axis=1, keepdims=True)
    hn = (h - mu) * jax.lax.rsqrt(var + 1e-5) * g_ref[...] + be_ref[...]
    reps = jnp.tanh(hn)
    bufp = buf_ref[...]
    rsq = reps * reps
    ones_row = jnp.ones((1, LATENT), jnp.float32)
    rrt = jax.lax.dot_general(ones_row, rsq, (((1,), (1,)), ((), ())),
                              preferred_element_type=jnp.float32)
    bb = jnp.sum(bufp * bufp, axis=1, keepdims=True)
    d2t = bb + rrt - 2.0 * jax.lax.dot_general(
        bufp, reps, (((1,), (1,)), ((), ())), preferred_element_type=jnp.float32)
    o_ref[...] = jnp.sqrt(jnp.maximum(d2t, 0.0))


def _topk_sc_body(dt_hbm, out_hbm, col_v, out_v):
    wid = lax.axis_index("s") * 2 + lax.axis_index("c")

    @pl.when(wid < 16)
    def _():
        pltpu.sync_copy(dt_hbm.at[wid], col_v)

        def body(j, carry):
            a, b, c = carry
            v = col_v[pl.ds(j * 16, 16)]
            t = jnp.maximum(a, v)
            a2 = jnp.minimum(a, v)
            t2 = jnp.maximum(b, t)
            b2 = jnp.minimum(b, t)
            c2 = jnp.minimum(c, t2)
            return a2, b2, c2

        big = jnp.full((16,), 1e30, jnp.float32)
        a, b, c = lax.fori_loop(0, BUFP, body, (big, big, big))
        out_v[...] = (a + b + c) * (1.0 / K)
        pltpu.sync_copy(out_v, out_hbm.at[pl.ds(wid * 16, 16)])


def _rew_body(m_ref, o_ref):
    o_ref[...] = -jnp.log(m_ref[...] + 1e-8)


def _ac_body(flat_ref, wh_ref, bh_ref, wa_ref, ba_ref, wv_ref, bv_ref,
             probs_ref, logp_ref, val_ref, acc_ref, *, nk):
    k = pl.program_id(0)

    @pl.when(k == 0)
    def _():
        acc_ref[...] = jnp.zeros_like(acc_ref)

    acc_ref[...] += jax.lax.dot_general(
        flat_ref[...], wh_ref[...], (((1,), (1,)), ((), ())),
        preferred_element_type=jnp.float32)

    @pl.when(k == nk - 1)
    def _():
        hid = jnp.maximum(acc_ref[...] + bh_ref[...], 0.0)
        logits = jax.lax.dot_general(hid, wa_ref[...], (((1,), (0,)), ((), ())),
                                     preferred_element_type=jnp.float32) + ba_ref[...]
        m = jnp.max(logits, axis=1, keepdims=True)
        e = jnp.exp(logits - m)
        s = jnp.sum(e, axis=1, keepdims=True)
        probs_ref[...] = e / s
        logp_ref[...] = logits - m - jnp.log(s)
        val_ref[...] = jax.lax.dot_general(
            hid, wv_ref[...], (((1,), (0,)), ((), ())),
            preferred_element_type=jnp.float32) + bv_ref[...]


def kernel(x, W1, b1, W2, b2, W3, b3, Wfc, bfc, gamma, beta, buffer,
           Wh, bh, Wa, ba, Wv, bv):
    flat = x.reshape(B, -1)
    kdim = flat.shape[1]
    nk = 4
    kc = kdim // nk
    wa_p = jnp.zeros((HID, NAP), jnp.float32).at[:, :NA].set(Wa.T)
    ba_p = jnp.full((1, NAP), -1e30, jnp.float32).at[:, :NA].set(ba)
    probs_p, logp_p, value = pl.pallas_call(
        functools.partial(_ac_body, nk=nk),
        grid=(nk,),
        in_specs=[
            pl.BlockSpec((B, kc), lambda k: (0, k)),
            pl.BlockSpec((HID, kc), lambda k: (0, k)),
            pl.BlockSpec((1, HID), lambda k: (0, 0)),
            pl.BlockSpec((HID, NAP), lambda k: (0, 0)),
            pl.BlockSpec((1, NAP), lambda k: (0, 0)),
            pl.BlockSpec((HID, 1), lambda k: (0, 0)),
            pl.BlockSpec((1, 1), lambda k: (0, 0)),
        ],
        out_specs=[
            pl.BlockSpec((B, NAP), lambda k: (0, 0)),
            pl.BlockSpec((B, NAP), lambda k: (0, 0)),
            pl.BlockSpec((B, 1), lambda k: (0, 0)),
        ],
        out_shape=[
            jax.ShapeDtypeStruct((B, NAP), jnp.float32),
            jax.ShapeDtypeStruct((B, NAP), jnp.float32),
            jax.ShapeDtypeStruct((B, 1), jnp.float32),
        ],
        scratch_shapes=[pltpu.VMEM((B, HID), jnp.float32)],
    )(flat, Wh, bh.reshape(1, HID), wa_p, ba_p, Wv.T, bv.reshape(1, 1))
    probs = probs_p[:, :NA]
    log_probs = logp_p[:, :NA]

    xp = jnp.pad(x, ((0, 0), (0, 0), (1, 3), (1, 15)))
    xq = xp.reshape(B, 4, G1, 4, G1C, 4).transpose(0, 2, 4, 3, 5, 1)
    xqf = xq.reshape(B, G1, G1C, 64)

    w1q = _pack_w1(W1)
    w2q = _pack_w23(W2, 32, 64)
    w3q = _pack_w23(W3, 64, 128)
    b1q = jnp.tile(b1, 4).reshape(1, 128)

    h3 = pl.pallas_call(
        _enc_body,
        grid=(B // BT,),
        in_specs=[
            pl.BlockSpec((BT, G1, G1C, 64), lambda i: (i, 0, 0, 0)),
            pl.BlockSpec((4, 64, 128), lambda i: (0, 0, 0)),
            pl.BlockSpec((4, 128, 64), lambda i: (0, 0, 0)),
            pl.BlockSpec((4, 256, 128), lambda i: (0, 0, 0)),
            pl.BlockSpec((1, 128), lambda i: (0, 0)),
            pl.BlockSpec((1, 64), lambda i: (0, 0)),
            pl.BlockSpec((1, 128), lambda i: (0, 0)),
        ],
        out_specs=pl.BlockSpec((BT, 10, 10, 128), lambda i: (i, 0, 0, 0)),
        out_shape=jax.ShapeDtypeStruct((B, 10, 10, 128), jnp.float32),
    )(xqf, w1q, w2q, w3q, b1q, b2.reshape(1, 64), b3.reshape(1, 128))

    h3f = h3.reshape(B, -1)
    wfc_r = Wfc.reshape(LATENT, 128, 10, 10).transpose(2, 3, 1, 0).reshape(-1, LATENT)
    buf_p = jnp.full((BUFP, LATENT), 1e3, jnp.float32).at[:BUF].set(buffer)

    dt = pl.pallas_call(
        _tail_body,
        in_specs=[
            pl.BlockSpec((B, 12800), lambda: (0, 0)),
            pl.BlockSpec((12800, LATENT), lambda: (0, 0)),
            pl.BlockSpec((1, LATENT), lambda: (0, 0)),
            pl.BlockSpec((1, LATENT), lambda: (0, 0)),
            pl.BlockSpec((1, LATENT), lambda: (0, 0)),
            pl.BlockSpec((BUFP, LATENT), lambda: (0, 0)),
        ],
        out_specs=pl.BlockSpec((BUFP, B), lambda: (0, 0)),
        out_shape=jax.ShapeDtypeStruct((BUFP, B), jnp.float32),
    )(h3f, wfc_r, bfc.reshape(1, LATENT), gamma.reshape(1, LATENT),
      beta.reshape(1, LATENT), buf_p)

    dt4 = dt.reshape(BUFP, 16, 16).transpose(1, 0, 2).reshape(16, BUFP * 16)
    mesh = plsc.VectorSubcoreMesh(core_axis_name="c", subcore_axis_name="s")
    knn = pl.kernel(
        _topk_sc_body, mesh=mesh,
        out_type=jax.ShapeDtypeStruct((B,), jnp.float32),
        scratch_types=[pltpu.VMEM((BUFP * 16,), jnp.float32),
                       pltpu.VMEM((16,), jnp.float32)],
    )(dt4)

    reward = pl.pallas_call(
        _rew_body,
        out_shape=jax.ShapeDtypeStruct((2, 128), jnp.float32),
    )(knn.reshape(2, 128))

    return (probs, log_probs, value, reward.reshape(B))


# speedup vs baseline: 1.0120x; 1.0120x over previous
import functools

import jax
import jax.numpy as jnp
from jax import lax
from jax.experimental import pallas as pl
from jax.experimental.pallas import tpu as pltpu, tpu_sc as plsc

B = 256
LATENT = 50
BUF = 500
BUFP = 512
K = 3
NA = 6
NAP = 8
HID = 512
BT = 16
G1 = 21
G1C = 24
G3 = 11
G3C = 16


def _enc_body(xq_ref, w1_ref, w2_ref, w3_ref, b1_ref, b2_ref, b3_ref, o_ref):
    xqm = xq_ref[...].reshape(BT * G1 * G1C, 64)
    d = [[None, None], [None, None]]
    for ia in range(2):
        for ib in range(2):
            d[ia][ib] = jax.lax.dot_general(
                xqm, w1_ref[ia * 2 + ib], (((1,), (0,)), ((), ())),
                preferred_element_type=jnp.float32).reshape(BT, G1, G1C, 128)
    y = d[1][1]
    y = y + jnp.pad(d[0][1][:, :G1 - 1], ((0, 0), (1, 0), (0, 0), (0, 0)))
    y = y + jnp.pad(d[1][0][:, :, :G1C - 1], ((0, 0), (0, 0), (1, 0), (0, 0)))
    y = y + jnp.pad(d[0][0][:, :G1 - 1, :G1C - 1],
                    ((0, 0), (1, 0), (1, 0), (0, 0)))
    y = jnp.maximum(y + b1_ref[...].reshape(1, 1, 1, 128), 0.0)
    ri = jax.lax.broadcasted_iota(jnp.int32, (1, G1, G1C, 128), 1)
    ci = jax.lax.broadcasted_iota(jnp.int32, (1, G1, G1C, 128), 2)
    li = jax.lax.broadcasted_iota(jnp.int32, (1, G1, G1C, 128), 3)
    lp = li // 64
    lq = (li // 32) % 2
    bad = ((lp == 0) & (ri == 0)) | ((lp == 1) & (ri == G1 - 1)) \
        | ((lq == 0) & (ci == 0)) | ((lq == 1) & (ci >= 20))
    xs2 = jnp.where(bad, 0.0, y)

    xs2m = xs2.reshape(BT * G1 * G1C, 128)
    for ia in range(2):
        for ib in range(2):
            d[ia][ib] = jax.lax.dot_general(
                xs2m, w2_ref[ia * 2 + ib], (((1,), (0,)), ((), ())),
                preferred_element_type=jnp.float32).reshape(BT, G1, G1C, 64)
    y2 = (d[0][0][:, :20, :20] + d[0][1][:, :20, 1:21]
          + d[1][0][:, 1:21, :20] + d[1][1][:, 1:21, 1:21])
    y2 = jnp.maximum(y2 + b2_ref[...].reshape(1, 1, 1, 64), 0.0)

    yr = y2.reshape(BT, 10, 2, 20, 64)
    er, orr = yr[:, :, 0], yr[:, :, 1]
    zrow = jnp.zeros((BT, 1, 20, 64), jnp.float32)
    r0 = jnp.concatenate([zrow, orr], axis=1)
    r1 = jnp.concatenate([er, zrow], axis=1)
    xs3p = []
    zc1 = jnp.zeros((BT, G3, 1, 64), jnp.float32)
    zc5 = jnp.zeros((BT, G3, 5, 64), jnp.float32)
    for rsel in (r0, r1):
        rc = rsel.reshape(BT, G3, 10, 2, 64)
        ec, oc = rc[:, :, :, 0], rc[:, :, :, 1]
        xs3p.append(jnp.concatenate([zc1, oc, zc5], axis=2))
        xs3p.append(jnp.concatenate([ec, zc1, zc5], axis=2))
    xs3 = jnp.concatenate(xs3p, axis=-1)

    xs3m = xs3.reshape(BT * G3 * G3C, 256)
    for ia in range(2):
        for ib in range(2):
            d[ia][ib] = jax.lax.dot_general(
                xs3m, w3_ref[ia * 2 + ib], (((1,), (0,)), ((), ())),
                preferred_element_type=jnp.float32).reshape(BT, G3, G3C, 128)
    y3 = (d[0][0][:, :10, :10] + d[0][1][:, :10, 1:11]
          + d[1][0][:, 1:11, :10] + d[1][1][:, 1:11, 1:11])
    y3 = jnp.maximum(y3 + b3_ref[...].reshape(1, 1, 1, 128), 0.0)
    o_ref[...] = y3


def _pack_w1(W1):
    ws = [[jnp.zeros((64, 128), jnp.float32) for _ in range(2)]
          for _ in range(2)]
    for P in range(2):
        for Q in range(2):
            for kh in range(3):
                dr = 2 * P + kh - 2
                ia = 1 if dr >= 0 else 0
                r4 = dr % 4
                for kw in range(3):
                    dc = 2 * Q + kw - 2
                    ib = 1 if dc >= 0 else 0
                    c4 = dc % 4
                    blk = W1[:, :, kh, kw].T
                    row = r4 * 16 + c4 * 4
                    col = (P * 2 + Q) * 32
                    ws[ia][ib] = jax.lax.dynamic_update_slice(
                        ws[ia][ib], blk, (row, col))
    return jnp.stack([ws[0][0], ws[0][1], ws[1][0], ws[1][1]])


def _pack_w23(W, cin, cout):
    ws = [[jnp.zeros((4 * cin, cout), jnp.float32) for _ in range(2)]
          for _ in range(2)]
    for kh in range(3):
        a, p = kh // 2, kh % 2 if kh < 2 else 0
        if kh == 2:
            a, p = 1, 0
        for kw in range(3):
            b, q = (1, 0) if kw == 2 else (0, kw)
            blk = W[:, :, kh, kw].T
            ws[a][b] = jax.lax.dynamic_update_slice(
                ws[a][b], blk, ((p * 2 + q) * cin, 0))
    return jnp.stack([ws[0][0], ws[0][1], ws[1][0], ws[1][1]])


def _tail_body(h3_ref, wfc_ref, bfc_ref, g_ref, be_ref, buf_ref, o_ref):
    h = jax.lax.dot_general(h3_ref[...], wfc_ref[...], (((1,), (0,)), ((), ())),
                            preferred_element_type=jnp.float32) + bfc_ref[...]
    mu = jnp.mean(h, axis=1, keepdims=True)
    var = jnp.mean((h - mu) * (h - mu), axis=1, keepdims=True)
    hn = (h - mu) * jax.lax.rsqrt(var + 1e-5) * g_ref[...] + be_ref[...]
    reps = jnp.tanh(hn)
    bufp = buf_ref[...]
    rsq = reps * reps
    ones_row = jnp.ones((1, LATENT), jnp.float32)
    rrt = jax.lax.dot_general(ones_row, rsq, (((1,), (1,)), ((), ())),
                              preferred_element_type=jnp.float32)
    bb = jnp.sum(bufp * bufp, axis=1, keepdims=True)
    d2t = bb + rrt - 2.0 * jax.lax.dot_general(
        bufp, reps, (((1,), (1,)), ((), ())), preferred_element_type=jnp.float32)
    o_ref[...] = jnp.sqrt(jnp.maximum(d2t, 0.0))


def _topk_sc_body(dt_hbm, out_hbm, col_v, out_v):
    wid = lax.axis_index("s") * 2 + lax.axis_index("c")

    @pl.when(wid < 16)
    def _():
        pltpu.sync_copy(dt_hbm.at[wid], col_v)

        def body(j, carry):
            a, b, c = carry
            v = col_v[pl.ds(j * 16, 16)]
            t = jnp.maximum(a, v)
            a2 = jnp.minimum(a, v)
            t2 = jnp.maximum(b, t)
            b2 = jnp.minimum(b, t)
            c2 = jnp.minimum(c, t2)
            return a2, b2, c2

        big = jnp.full((16,), 1e30, jnp.float32)
        a, b, c = lax.fori_loop(0, BUFP, body, (big, big, big))
        out_v[...] = (a + b + c) * (1.0 / K)
        pltpu.sync_copy(out_v, out_hbm.at[pl.ds(wid * 16, 16)])


def _rew_body(m_ref, o_ref):
    o_ref[...] = -jnp.log(m_ref[...] + 1e-8)


def _ac_body(flat_ref, wh_ref, bh_ref, wa_ref, ba_ref, wv_ref, bv_ref,
             probs_ref, logp_ref, val_ref, acc_ref, *, nk):
    k = pl.program_id(0)

    @pl.when(k == 0)
    def _():
        acc_ref[...] = jnp.zeros_like(acc_ref)

    acc_ref[...] += jax.lax.dot_general(
        flat_ref[...], wh_ref[...], (((1,), (1,)), ((), ())),
        preferred_element_type=jnp.float32)

    @pl.when(k == nk - 1)
    def _():
        hid = jnp.maximum(acc_ref[...] + bh_ref[...], 0.0)
        logits = jax.lax.dot_general(hid, wa_ref[...], (((1,), (0,)), ((), ())),
                                     preferred_element_type=jnp.float32) + ba_ref[...]
        m = jnp.max(logits, axis=1, keepdims=True)
        e = jnp.exp(logits - m)
        s = jnp.sum(e, axis=1, keepdims=True)
        probs_ref[...] = e / s
        logp_ref[...] = logits - m - jnp.log(s)
        val_ref[...] = jax.lax.dot_general(
            hid, wv_ref[...], (((1,), (0,)), ((), ())),
            preferred_element_type=jnp.float32) + bv_ref[...]


def kernel(x, W1, b1, W2, b2, W3, b3, Wfc, bfc, gamma, beta, buffer,
           Wh, bh, Wa, ba, Wv, bv):
    flat = x.reshape(B, -1)
    kdim = flat.shape[1]
    nk = 8
    kc = kdim // nk
    wa_p = jnp.zeros((HID, NAP), jnp.float32).at[:, :NA].set(Wa.T)
    ba_p = jnp.full((1, NAP), -1e30, jnp.float32).at[:, :NA].set(ba)
    probs_p, logp_p, value = pl.pallas_call(
        functools.partial(_ac_body, nk=nk),
        grid=(nk,),
        in_specs=[
            pl.BlockSpec((B, kc), lambda k: (0, k)),
            pl.BlockSpec((HID, kc), lambda k: (0, k)),
            pl.BlockSpec((1, HID), lambda k: (0, 0)),
            pl.BlockSpec((HID, NAP), lambda k: (0, 0)),
            pl.BlockSpec((1, NAP), lambda k: (0, 0)),
            pl.BlockSpec((HID, 1), lambda k: (0, 0)),
            pl.BlockSpec((1, 1), lambda k: (0, 0)),
        ],
        out_specs=[
            pl.BlockSpec((B, NAP), lambda k: (0, 0)),
            pl.BlockSpec((B, NAP), lambda k: (0, 0)),
            pl.BlockSpec((B, 1), lambda k: (0, 0)),
        ],
        out_shape=[
            jax.ShapeDtypeStruct((B, NAP), jnp.float32),
            jax.ShapeDtypeStruct((B, NAP), jnp.float32),
            jax.ShapeDtypeStruct((B, 1), jnp.float32),
        ],
        scratch_shapes=[pltpu.VMEM((B, HID), jnp.float32)],
    )(flat, Wh, bh.reshape(1, HID), wa_p, ba_p, Wv.T, bv.reshape(1, 1))
    probs = probs_p[:, :NA]
    log_probs = logp_p[:, :NA]

    xp = jnp.pad(x, ((0, 0), (0, 0), (1, 3), (1, 15)))
    xq = xp.reshape(B, 4, G1, 4, G1C, 4).transpose(0, 2, 4, 3, 5, 1)
    xqf = xq.reshape(B, G1, G1C, 64)

    w1q = _pack_w1(W1)
    w2q = _pack_w23(W2, 32, 64)
    w3q = _pack_w23(W3, 64, 128)
    b1q = jnp.tile(b1, 4).reshape(1, 128)

    h3 = pl.pallas_call(
        _enc_body,
        grid=(B // BT,),
        in_specs=[
            pl.BlockSpec((BT, G1, G1C, 64), lambda i: (i, 0, 0, 0)),
            pl.BlockSpec((4, 64, 128), lambda i: (0, 0, 0)),
            pl.BlockSpec((4, 128, 64), lambda i: (0, 0, 0)),
            pl.BlockSpec((4, 256, 128), lambda i: (0, 0, 0)),
            pl.BlockSpec((1, 128), lambda i: (0, 0)),
            pl.BlockSpec((1, 64), lambda i: (0, 0)),
            pl.BlockSpec((1, 128), lambda i: (0, 0)),
        ],
        out_specs=pl.BlockSpec((BT, 10, 10, 128), lambda i: (i, 0, 0, 0)),
        out_shape=jax.ShapeDtypeStruct((B, 10, 10, 128), jnp.float32),
    )(xqf, w1q, w2q, w3q, b1q, b2.reshape(1, 64), b3.reshape(1, 128))

    h3f = h3.reshape(B, -1)
    wfc_r = Wfc.reshape(LATENT, 128, 10, 10).transpose(2, 3, 1, 0).reshape(-1, LATENT)
    buf_p = jnp.full((BUFP, LATENT), 1e3, jnp.float32).at[:BUF].set(buffer)

    dt = pl.pallas_call(
        _tail_body,
        in_specs=[
            pl.BlockSpec((B, 12800), lambda: (0, 0)),
            pl.BlockSpec((12800, LATENT), lambda: (0, 0)),
            pl.BlockSpec((1, LATENT), lambda: (0, 0)),
            pl.BlockSpec((1, LATENT), lambda: (0, 0)),
            pl.BlockSpec((1, LATENT), lambda: (0, 0)),
            pl.BlockSpec((BUFP, LATENT), lambda: (0, 0)),
        ],
        out_specs=pl.BlockSpec((BUFP, B), lambda: (0, 0)),
        out_shape=jax.ShapeDtypeStruct((BUFP, B), jnp.float32),
    )(h3f, wfc_r, bfc.reshape(1, LATENT), gamma.reshape(1, LATENT),
      beta.reshape(1, LATENT), buf_p)

    dt4 = dt.reshape(BUFP, 16, 16).transpose(1, 0, 2).reshape(16, BUFP * 16)
    mesh = plsc.VectorSubcoreMesh(core_axis_name="c", subcore_axis_name="s")
    knn = pl.kernel(
        _topk_sc_body, mesh=mesh,
        out_type=jax.ShapeDtypeStruct((B,), jnp.float32),
        scratch_types=[pltpu.VMEM((BUFP * 16,), jnp.float32),
                       pltpu.VMEM((16,), jnp.float32)],
    )(dt4)

    reward = pl.pallas_call(
        _rew_body,
        out_shape=jax.ShapeDtypeStruct((2, 128), jnp.float32),
    )(knn.reshape(2, 128))

    return (probs, log_probs, value, reward.reshape(B))
